# baseline (speedup 1.0000x reference)
import functools

import jax
import jax.numpy as jnp
from jax import lax
from jax.experimental import pallas as pl
from jax.experimental.pallas import tpu as pltpu
from jax.experimental.pallas import tpu_sc as plsc

DIM = 64
BATCH = 16384
NC = 2
NS = 16
L = 16
NW = NC * NS
BPW = BATCH // NW
C = 128
NCH = BPW // C
VPR = DIM // L

_mesh = plsc.VectorSubcoreMesh(
    core_axis_name="c", subcore_axis_name="s", num_cores=NC, num_subcores=NS
)


@functools.partial(
    pl.kernel,
    out_type=jax.ShapeDtypeStruct((NW, 2, L), jnp.float32),
    mesh=_mesh,
    scratch_types=[
        pltpu.VMEM((NCH, C), jnp.int32),
        pltpu.VMEM((NCH, C), jnp.int32),
        pltpu.VMEM((NCH, C), jnp.int32),
        pltpu.VMEM((NCH, C), jnp.int32),
        pltpu.VMEM((C, DIM), jnp.float32),
        pltpu.VMEM((C, DIM), jnp.float32),
        pltpu.VMEM((C, DIM), jnp.float32),
        pltpu.VMEM((C, DIM), jnp.float32),
        pltpu.VMEM((2, L), jnp.float32),
        pltpu.SemaphoreType.DMA,
    ],
    compiler_params=pltpu.CompilerParams(use_tc_tiling_on_sc=False),
)
def _w2v_partials(iw_hbm, ow_hbm, inw_hbm, onw_hbm, itab_hbm, otab_hbm,
                  out_hbm, idx_i, idx_o, idx_in, idx_on, ri, ro, rin, ron,
                  ob, sem):
    wid = lax.axis_index("s") * NC + lax.axis_index("c")

    pltpu.sync_copy(iw_hbm.at[wid], idx_i)
    pltpu.sync_copy(ow_hbm.at[wid], idx_o)
    pltpu.sync_copy(inw_hbm.at[wid], idx_in)
    pltpu.sync_copy(onw_hbm.at[wid], idx_on)

    zeros = jnp.zeros((L,), jnp.float32)
    acc1 = [zeros] * VPR
    acc2 = [zeros] * VPR

    for j in range(NCH):
        cps = [
            pltpu.async_copy(itab_hbm.at[idx_i.at[j]], ri, sem),
            pltpu.async_copy(otab_hbm.at[idx_o.at[j]], ro, sem),
            pltpu.async_copy(itab_hbm.at[idx_in.at[j]], rin, sem),
            pltpu.async_copy(otab_hbm.at[idx_on.at[j]], ron, sem),
        ]
        for cp in cps:
            cp.wait()

        def row_body(r, accs):
            a1, a2 = accs
            a1 = tuple(
                a1[p] + ri[r, pl.ds(p * L, L)] * ro[r, pl.ds(p * L, L)]
                for p in range(VPR)
            )
            a2 = tuple(
                a2[p] + rin[r, pl.ds(p * L, L)] * ron[r, pl.ds(p * L, L)]
                for p in range(VPR)
            )
            return (a1, a2)

        acc1, acc2 = lax.fori_loop(0, C, row_body, (tuple(acc1), tuple(acc2)))

    t1 = acc1[0] + acc1[1] + acc1[2] + acc1[3]
    t2 = acc2[0] + acc2[1] + acc2[2] + acc2[3]
    ob[0, :] = t1
    ob[1, :] = t2
    pltpu.sync_copy(ob, out_hbm.at[wid])


def kernel(iword, oword, inword, onword, ivectors_table, ovectors_table):
    iw = iword.reshape(NW, NCH, C)
    ow = oword.reshape(NW, NCH, C)
    inw = inword.reshape(NW, NCH, C)
    onw = onword.reshape(NW, NCH, C)
    parts = _w2v_partials(iw, ow, inw, onw, ivectors_table, ovectors_table)
    s1 = parts[:, 0, :].sum()
    s2 = parts[:, 1, :].sum()
    oloss = jnp.log(jnp.clip(jax.nn.sigmoid(s1), 1e-12, 1.0))
    nloss = jnp.log(jnp.clip(jax.nn.sigmoid(-s2), 1e-12, 1.0))
    return -(oloss + nloss)
